# Initial kernel scaffold; baseline (speedup 1.0000x reference)
import jax
import jax.numpy as jnp
from jax.experimental import pallas as pl


def kernel(features_0, edge_index, e_feat_org, fc_w, fc_b, ln0_g, ln0_b, w1, b1, ln1_g, ln1_b, w2, b2):
    raise NotImplementedError("write your pallas kernel here")


# speedup vs baseline: 3.0332x; 3.0332x over previous
import functools

import jax
import jax.numpy as jnp
from jax import lax
from jax.experimental import pallas as pl
from jax.experimental.pallas import tpu as pltpu
from jax.experimental.pallas import tpu_sc as plsc

N = 10000
E = 320000
D_IN = 128
H = 128
C_OUT = 64
L = 4

NC = 2
NS = 16
NW = NC * NS

CHUNK = 128
CHUNKS_PER_TILE = 80
EPT = CHUNK * CHUNKS_PER_TILE
E_PAD = EPT * NW
TRASH = N
NP16 = 10112
ROWS_PER_TILE = NP16 // NS

BLK = 2000
GRID = N // BLK

_mesh = plsc.VectorSubcoreMesh(core_axis_name="c", subcore_axis_name="s")


def _sc_deg128_body(idxv, deg_out, idx_d, ones_v, acc_sh):
    c = lax.axis_index("c")
    s = lax.axis_index("s")
    wid = s * NC + c

    pltpu.sync_copy(idxv.at[pl.ds(wid * CHUNKS_PER_TILE, CHUNKS_PER_TILE)], idx_d)

    one16 = jnp.ones((16,), jnp.float32)
    zro16 = jnp.zeros((16,), jnp.float32)

    @pl.loop(0, CHUNK)
    def _fill0(i):
        for k in range(H // 16):
            ones_v[i, pl.ds(k * 16, 16)] = zro16

    base = s * ROWS_PER_TILE
    for k in range(5):
        lo = k * CHUNK
        nrows = min(CHUNK, ROWS_PER_TILE - lo)
        pltpu.sync_copy(ones_v.at[pl.ds(0, nrows)],
                        acc_sh.at[pl.ds(base + lo, nrows)])

    @pl.loop(0, CHUNK)
    def _fill1(i):
        ones_v[i, pl.ds(0, 16)] = one16

    plsc.subcore_barrier()

    @pl.loop(0, CHUNKS_PER_TILE)
    def _scatter(j):
        pltpu.sync_copy(ones_v, acc_sh.at[idx_d.at[j]], add=True)

    plsc.subcore_barrier()
    pltpu.sync_copy(acc_sh.at[pl.ds(base, ROWS_PER_TILE)],
                    deg_out.at[c, pl.ds(base, ROWS_PER_TILE)])


_sc_deg128 = functools.partial(
    pl.kernel,
    out_type=jax.ShapeDtypeStruct((NC, NP16, H), jnp.float32),
    mesh=_mesh,
    scratch_types=[
        pltpu.VMEM((CHUNKS_PER_TILE, CHUNK), jnp.int32),
        pltpu.VMEM((CHUNK, H), jnp.float32),
        pltpu.VMEM_SHARED((NP16, H), jnp.float32),
    ],
)(_sc_deg128_body)


HALF = CHUNKS_PER_TILE // 2


def _sc_spmm_body(t_hbm, srcv, dstv, a_out,
                  idx_s, idx_d, rows0, rows1, acc_sh,
                  gsem0, gsem1):
    c = lax.axis_index("c")
    s = lax.axis_index("s")
    wid = s * NC + c

    zro16 = jnp.zeros((16,), jnp.float32)

    @pl.loop(0, CHUNK)
    def _fill(i):
        for k in range(H // 16):
            rows0[i, pl.ds(k * 16, 16)] = zro16

    base = s * ROWS_PER_TILE
    for k in range(5):
        lo = k * CHUNK
        nrows = min(CHUNK, ROWS_PER_TILE - lo)
        pltpu.sync_copy(rows0.at[pl.ds(0, nrows)],
                        acc_sh.at[pl.ds(base + lo, nrows)])
    plsc.subcore_barrier()

    for half in range(2):
        cbase = wid * CHUNKS_PER_TILE + half * HALF
        pltpu.sync_copy(srcv.at[pl.ds(cbase, HALF)], idx_s)
        pltpu.sync_copy(dstv.at[pl.ds(cbase, HALF)], idx_d)

        pltpu.make_async_copy(t_hbm.at[idx_s.at[0]], rows0, gsem0).start()
        pltpu.make_async_copy(t_hbm.at[idx_s.at[1]], rows1, gsem1).start()

        @pl.loop(0, HALF, step=2)
        def _edges(j0):
            for b, (rows, gsem) in enumerate(((rows0, gsem0), (rows1, gsem1))):
                j = j0 + b
                pltpu.make_async_copy(t_hbm.at[idx_s.at[j]], rows, gsem).wait()
                pltpu.sync_copy(rows, acc_sh.at[idx_d.at[j]], add=True)

                @pl.when(j + 2 < HALF)
                def _():
                    pltpu.make_async_copy(
                        t_hbm.at[idx_s.at[j + 2]], rows, gsem).start()

    plsc.subcore_barrier()
    pltpu.sync_copy(acc_sh.at[pl.ds(base, ROWS_PER_TILE)],
                    a_out.at[c, pl.ds(base, ROWS_PER_TILE)])


_sc_spmm = functools.partial(
    pl.kernel,
    out_type=jax.ShapeDtypeStruct((NC, NP16, H), jnp.float32),
    mesh=_mesh,
    scratch_types=[
        pltpu.VMEM((HALF, CHUNK), jnp.int32),
        pltpu.VMEM((HALF, CHUNK), jnp.int32),
        pltpu.VMEM((CHUNK, H), jnp.float32),
        pltpu.VMEM((CHUNK, H), jnp.float32),
        pltpu.VMEM_SHARED((NP16, H), jnp.float32),
        pltpu.SemaphoreType.DMA,
        pltpu.SemaphoreType.DMA,
    ],
)(_sc_spmm_body)


def _tc_prep_body(feat_ref, fcw_ref, fcb_ref, dego_ref, degi_ref,
                  h0_ref, t0_ref, ni_ref, c_ref):
    deg_o = (dego_ref[0] + dego_ref[1])[:, 0:1]
    deg_i = (degi_ref[0] + degi_ref[1])[:, 0:1]
    no = lax.rsqrt(jnp.maximum(deg_o, 1.0))
    ni = lax.rsqrt(jnp.maximum(deg_i, 1.0))
    h0 = jnp.dot(feat_ref[...], fcw_ref[...],
                 preferred_element_type=jnp.float32) + fcb_ref[...]
    h0_ref[...] = h0
    t0_ref[...] = h0 * no
    ni_ref[...] = jnp.broadcast_to(ni, (BLK, 16))
    c_ref[...] = jnp.broadcast_to(ni * no, (BLK, 16))


def _tc_prep(feat, fc_w, fc_b, degp_o, degp_i):
    return pl.pallas_call(
        _tc_prep_body,
        grid=(GRID,),
        in_specs=[
            pl.BlockSpec((BLK, D_IN), lambda i: (i, 0)),
            pl.BlockSpec((D_IN, H), lambda i: (0, 0)),
            pl.BlockSpec((1, H), lambda i: (0, 0)),
            pl.BlockSpec((NC, BLK, H), lambda i: (0, i, 0)),
            pl.BlockSpec((NC, BLK, H), lambda i: (0, i, 0)),
        ],
        out_specs=[
            pl.BlockSpec((BLK, H), lambda i: (i, 0)),
            pl.BlockSpec((BLK, H), lambda i: (i, 0)),
            pl.BlockSpec((BLK, 16), lambda i: (i, 0)),
            pl.BlockSpec((BLK, 16), lambda i: (i, 0)),
        ],
        out_shape=[
            jax.ShapeDtypeStruct((N, H), jnp.float32),
            jax.ShapeDtypeStruct((NP16, H), jnp.float32),
            jax.ShapeDtypeStruct((N, 16), jnp.float32),
            jax.ShapeDtypeStruct((N, 16), jnp.float32),
        ],
    )(feat, fc_w, fc_b, degp_o, degp_i)


def _tc_scale_body(ap_ref, hs_ref, ni_ref, c_ref, hs_out, t_out):
    asum = ap_ref[0] + ap_ref[1]
    hs_out[...] = hs_ref[...] + asum * ni_ref[:, 0:1]
    t_out[...] = asum * c_ref[:, 0:1]


def _tc_scale(a_parts, hsum, ni, cc):
    return pl.pallas_call(
        _tc_scale_body,
        grid=(GRID,),
        in_specs=[
            pl.BlockSpec((NC, BLK, H), lambda i: (0, i, 0)),
            pl.BlockSpec((BLK, H), lambda i: (i, 0)),
            pl.BlockSpec((BLK, 16), lambda i: (i, 0)),
            pl.BlockSpec((BLK, 16), lambda i: (i, 0)),
        ],
        out_specs=[
            pl.BlockSpec((BLK, H), lambda i: (i, 0)),
            pl.BlockSpec((BLK, H), lambda i: (i, 0)),
        ],
        out_shape=[
            jax.ShapeDtypeStruct((N, H), jnp.float32),
            jax.ShapeDtypeStruct((NP16, H), jnp.float32),
        ],
    )(a_parts, hsum, ni, cc)


def _tc_head_body(hs_ref, g0_ref, b0_ref, w1_ref, b1_ref, g1_ref, b1n_ref,
                  w2_ref, b2_ref, out_ref):
    x = hs_ref[...] * (1.0 / (L + 1.0))
    mu = jnp.mean(x, axis=1, keepdims=True)
    var = jnp.mean((x - mu) * (x - mu), axis=1, keepdims=True)
    x = (x - mu) * lax.rsqrt(var + 1e-5) * g0_ref[...] + b0_ref[...]
    y = jnp.dot(x, w1_ref[...], preferred_element_type=jnp.float32) + b1_ref[...]
    y = jnp.where(y > 0, y, jnp.exp(jnp.minimum(y, 0.0)) - 1.0)
    mu = jnp.mean(y, axis=1, keepdims=True)
    var = jnp.mean((y - mu) * (y - mu), axis=1, keepdims=True)
    y = (y - mu) * lax.rsqrt(var + 1e-5) * g1_ref[...] + b1n_ref[...]
    z = jnp.dot(y, w2_ref[...], preferred_element_type=jnp.float32) + b2_ref[...]
    nrm = jnp.sqrt(jnp.sum(z * z, axis=1, keepdims=True))
    out_ref[...] = z / jnp.maximum(nrm, 1e-12)


def _tc_head(hsum, ln0_g, ln0_b, w1, b1, ln1_g, ln1_b, w2, b2):
    return pl.pallas_call(
        _tc_head_body,
        grid=(GRID,),
        in_specs=[
            pl.BlockSpec((BLK, H), lambda i: (i, 0)),
            pl.BlockSpec((1, H), lambda i: (0, 0)),
            pl.BlockSpec((1, H), lambda i: (0, 0)),
            pl.BlockSpec((H, H), lambda i: (0, 0)),
            pl.BlockSpec((1, H), lambda i: (0, 0)),
            pl.BlockSpec((1, H), lambda i: (0, 0)),
            pl.BlockSpec((1, H), lambda i: (0, 0)),
            pl.BlockSpec((H, C_OUT), lambda i: (0, 0)),
            pl.BlockSpec((1, C_OUT), lambda i: (0, 0)),
        ],
        out_specs=pl.BlockSpec((BLK, C_OUT), lambda i: (i, 0)),
        out_shape=jax.ShapeDtypeStruct((N, C_OUT), jnp.float32),
    )(hsum, ln0_g, ln0_b, w1, b1, ln1_g, ln1_b, w2, b2)


def kernel(features_0, edge_index, e_feat_org, fc_w, fc_b, ln0_g, ln0_b,
           w1, b1, ln1_g, ln1_b, w2, b2):
    src = edge_index[0]
    dst = edge_index[1]
    pad = E_PAD - E
    fill = jnp.full((pad,), TRASH, jnp.int32)
    srcv = jnp.concatenate([src, fill]).reshape(NW * CHUNKS_PER_TILE, CHUNK)
    dstv = jnp.concatenate([dst, fill]).reshape(NW * CHUNKS_PER_TILE, CHUNK)

    degp_o = _sc_deg128(srcv)
    degp_i = _sc_deg128(dstv)
    hsum, t, ni, cc = _tc_prep(features_0, fc_w, fc_b.reshape(1, H),
                               degp_o, degp_i)
    for _ in range(L):
        a_parts = _sc_spmm(t, srcv, dstv)
        hsum, t = _tc_scale(a_parts, hsum, ni, cc)

    return _tc_head(hsum, ln0_g.reshape(1, H), ln0_b.reshape(1, H),
                    w1, b1.reshape(1, H), ln1_g.reshape(1, H),
                    ln1_b.reshape(1, H), w2, b2.reshape(1, C_OUT))


# speedup vs baseline: 4.3942x; 1.4487x over previous
import functools

import jax
import jax.numpy as jnp
from jax import lax
from jax.experimental import pallas as pl
from jax.experimental.pallas import tpu as pltpu
from jax.experimental.pallas import tpu_sc as plsc

N = 10000
E = 320000
D_IN = 128
H = 128
C_OUT = 64
L = 4

NC = 2
NS = 16
NW = NC * NS

COLS = H // NC
ECH = 64
CPT = 320
QCH = 64
E_PAD = ECH * CPT * NS
NP16 = 10112
TROWS = NP16 // NS

BLK = 2000
BLKF = TROWS
GRID = N // BLK

_mesh = plsc.VectorSubcoreMesh(core_axis_name="c", subcore_axis_name="s")


def _sc_mega_body(h0p, srcv, dstv, hsum_out, t_flat,
                  idx_s, idx_d, rows0, rows1, no_v, ni_v, cc_v,
                  acc_sh, gsem0, gsem1):
    c = lax.axis_index("c")
    s = lax.axis_index("s")
    base = s * TROWS
    NVH = H // 16

    zro16 = jnp.zeros((16,), jnp.float32)
    one16 = jnp.ones((16,), jnp.float32)
    mask0 = lax.broadcasted_iota(jnp.int32, (16,), 0) == 0

    def fill(buf, val16):
        @pl.loop(0, 64)
        def _f(i):
            for k in range(NVH):
                buf[i, pl.ds(k * 16, 16)] = val16

    def zero_acc_slice():
        @pl.loop(0, 9)
        def _z(k):
            pltpu.sync_copy(rows0.at[pl.ds(0, 64)],
                            acc_sh.at[pl.ds(base + k * 64, 64)])
        pltpu.sync_copy(rows0.at[pl.ds(0, 56)],
                        acc_sh.at[pl.ds(base + 576, 56)])

    def norm_block(lo, nr, p):
        pltpu.sync_copy(acc_sh.at[pl.ds(base + lo, nr)],
                        rows0.at[pl.ds(0, nr)])

        @pl.loop(0, nr)
        def _n(r):
            x = jnp.maximum(rows0[r, pl.ds(0, 16)], 1.0)
            iv = plsc.bitcast(x, jnp.int32)
            y = plsc.bitcast(jnp.int32(0x5F3759DF) - (iv >> 1), jnp.float32)
            for _ in range(3):
                y = y * (1.5 - 0.5 * x * y * y)
            ridx = jnp.full((16,), lo + r, jnp.int32)
            if p == 0:
                plsc.store_scatter(no_v, [ridx], y, mask=mask0)
            else:
                plsc.store_scatter(ni_v, [ridx], y, mask=mask0)
                nov = plsc.load_gather(no_v, [ridx])
                plsc.store_scatter(cc_v, [ridx], y * nov, mask=mask0)

    def stage_block(lo, nr):
        pltpu.sync_copy(h0p.at[pl.ds(base + lo, nr)], rows0.at[pl.ds(0, nr)])
        pltpu.sync_copy(rows0.at[pl.ds(0, nr)],
                        hsum_out.at[c, pl.ds(base + lo, nr)])

        @pl.loop(0, nr)
        def _t0(r):
            nov = plsc.load_gather(no_v, [jnp.full((16,), lo + r, jnp.int32)])
            for kk in range(NVH):
                sl = pl.ds(kk * 16, 16)
                rows0[r, sl] = rows0[r, sl] * nov

        pltpu.sync_copy(rows0.at[pl.ds(0, nr)],
                        t_flat.at[pl.ds(c * NP16 + base + lo, nr)])

    def edge_pass():
        coff = c * NP16

        @pl.loop(0, CPT // QCH)
        def _q(q):
            hb = s * CPT + q * QCH
            pltpu.sync_copy(srcv.at[pl.ds(hb, QCH)], idx_s)
            pltpu.sync_copy(dstv.at[pl.ds(hb, QCH)], idx_d)

            @pl.loop(0, QCH)
            def _off(r):
                for kk in range(ECH // 16):
                    sl = pl.ds(kk * 16, 16)
                    idx_s[r, sl] = idx_s[r, sl] + coff

            pltpu.make_async_copy(t_flat.at[idx_s.at[0]], rows0, gsem0).start()
            pltpu.make_async_copy(t_flat.at[idx_s.at[1]], rows1, gsem1).start()

            @pl.loop(0, QCH, step=2)
            def _edges(j0):
                for b, (rows, gsem) in enumerate(
                        ((rows0, gsem0), (rows1, gsem1))):
                    j = j0 + b
                    pltpu.make_async_copy(
                        t_flat.at[idx_s.at[j]], rows, gsem).wait()
                    pltpu.sync_copy(rows, acc_sh.at[idx_d.at[j]], add=True)

                    @pl.when(j + 2 < QCH)
                    def _():
                        pltpu.make_async_copy(
                            t_flat.at[idx_s.at[j + 2]], rows, gsem).start()

    def scale_block(lo, nr, last):
        pltpu.sync_copy(acc_sh.at[pl.ds(base + lo, nr)],
                        rows0.at[pl.ds(0, nr)])
        pltpu.sync_copy(hsum_out.at[c, pl.ds(base + lo, nr)],
                        rows1.at[pl.ds(0, nr)])

        if last:
            @pl.loop(0, nr)
            def _upd_last(r):
                ridx = jnp.full((16,), lo + r, jnp.int32)
                niv = plsc.load_gather(ni_v, [ridx])
                for kk in range(NVH):
                    sl = pl.ds(kk * 16, 16)
                    rows1[r, sl] = rows1[r, sl] + niv * rows0[r, sl]
        else:
            @pl.loop(0, nr)
            def _upd(r):
                ridx = jnp.full((16,), lo + r, jnp.int32)
                niv = plsc.load_gather(ni_v, [ridx])
                ccv = plsc.load_gather(cc_v, [ridx])
                for kk in range(NVH):
                    sl = pl.ds(kk * 16, 16)
                    a = rows0[r, sl]
                    rows1[r, sl] = rows1[r, sl] + niv * a
                    rows0[r, sl] = ccv * a

        pltpu.sync_copy(rows1.at[pl.ds(0, nr)],
                        hsum_out.at[c, pl.ds(base + lo, nr)])
        if not last:
            pltpu.sync_copy(rows0.at[pl.ds(0, nr)],
                            t_flat.at[pl.ds(c * NP16 + base + lo, nr)])

    def scale_phase(last):
        @pl.loop(0, 9)
        def _sc(k):
            scale_block(k * 64, 64, last)

        scale_block(576, 56, last)
        if not last:
            fill(rows0, zro16)
            zero_acc_slice()

    fill(rows0, zro16)
    zero_acc_slice()
    plsc.subcore_barrier()

    for p in range(2):
        idxv = srcv if p == 0 else dstv
        fill(rows1, one16)

        @pl.loop(0, CPT // QCH)
        def _dq(q):
            pltpu.sync_copy(idxv.at[pl.ds(s * CPT + q * QCH, QCH)], idx_d)

            @pl.loop(0, QCH)
            def _deg_scatter(j):
                pltpu.sync_copy(rows1.at[pl.ds(0, ECH)],
                                acc_sh.at[idx_d.at[j]], add=True)

        plsc.subcore_barrier()

        @pl.loop(0, 9)
        def _nb(k):
            norm_block(k * 64, 64, p)

        norm_block(576, 56, p)

        fill(rows0, zro16)
        zero_acc_slice()
        plsc.subcore_barrier()

    @pl.loop(0, 9)
    def _stg(k):
        stage_block(k * 64, 64)

    stage_block(576, 56)
    plsc.subcore_barrier()

    @pl.loop(0, L - 1)
    def _layer(_):
        edge_pass()
        plsc.subcore_barrier()
        scale_phase(False)
        plsc.subcore_barrier()

    edge_pass()
    plsc.subcore_barrier()
    scale_phase(True)


_sc_mega = functools.partial(
    pl.kernel,
    out_type=[jax.ShapeDtypeStruct((NC, NP16, H), jnp.float32),
              jax.ShapeDtypeStruct((NC * NP16, H), jnp.float32)],
    mesh=_mesh,
    scratch_types=[
        pltpu.VMEM((QCH, ECH), jnp.int32),
        pltpu.VMEM((QCH, ECH), jnp.int32),
        pltpu.VMEM((ECH, H), jnp.float32),
        pltpu.VMEM((ECH, H), jnp.float32),
        pltpu.VMEM((TROWS,), jnp.float32),
        pltpu.VMEM((TROWS,), jnp.float32),
        pltpu.VMEM((TROWS,), jnp.float32),
        pltpu.VMEM_SHARED((NP16, H), jnp.float32),
        pltpu.SemaphoreType.DMA,
        pltpu.SemaphoreType.DMA,
    ],
    compiler_params=pltpu.CompilerParams(needs_layout_passes=False),
)(_sc_mega_body)


def _tc_fc_body(feat_ref, fcw_ref, fcb_ref, h0_ref):
    h0_ref[...] = jnp.dot(feat_ref[...], fcw_ref[...],
                          preferred_element_type=jnp.float32) + fcb_ref[...]


def _tc_fc(feat_pad, fc_w, fc_b):
    return pl.pallas_call(
        _tc_fc_body,
        grid=(NP16 // BLKF,),
        in_specs=[
            pl.BlockSpec((BLKF, D_IN), lambda i: (i, 0)),
            pl.BlockSpec((D_IN, H), lambda i: (0, 0)),
            pl.BlockSpec((1, H), lambda i: (0, 0)),
        ],
        out_specs=pl.BlockSpec((BLKF, H), lambda i: (i, 0)),
        out_shape=jax.ShapeDtypeStruct((NP16, H), jnp.float32),
    )(feat_pad, fc_w, fc_b)


def _tc_head_body(hs_ref, g0_ref, b0_ref, w1_ref, b1_ref, g1_ref, b1n_ref,
                  w2_ref, b2_ref, out_ref):
    hs = jnp.concatenate([hs_ref[0][:, :COLS], hs_ref[1][:, COLS:]], axis=1)
    x = hs * (1.0 / (L + 1.0))
    mu = jnp.mean(x, axis=1, keepdims=True)
    var = jnp.mean((x - mu) * (x - mu), axis=1, keepdims=True)
    x = (x - mu) * lax.rsqrt(var + 1e-5) * g0_ref[...] + b0_ref[...]
    y = jnp.dot(x, w1_ref[...], preferred_element_type=jnp.float32) + b1_ref[...]
    y = jnp.where(y > 0, y, jnp.exp(jnp.minimum(y, 0.0)) - 1.0)
    mu = jnp.mean(y, axis=1, keepdims=True)
    var = jnp.mean((y - mu) * (y - mu), axis=1, keepdims=True)
    y = (y - mu) * lax.rsqrt(var + 1e-5) * g1_ref[...] + b1n_ref[...]
    z = jnp.dot(y, w2_ref[...], preferred_element_type=jnp.float32) + b2_ref[...]
    nrm = jnp.sqrt(jnp.sum(z * z, axis=1, keepdims=True))
    out_ref[...] = z / jnp.maximum(nrm, 1e-12)


def _tc_head(hsum2, ln0_g, ln0_b, w1, b1, ln1_g, ln1_b, w2, b2):
    return pl.pallas_call(
        _tc_head_body,
        grid=(GRID,),
        in_specs=[
            pl.BlockSpec((NC, BLK, H), lambda i: (0, i, 0)),
            pl.BlockSpec((1, H), lambda i: (0, 0)),
            pl.BlockSpec((1, H), lambda i: (0, 0)),
            pl.BlockSpec((H, H), lambda i: (0, 0)),
            pl.BlockSpec((1, H), lambda i: (0, 0)),
            pl.BlockSpec((1, H), lambda i: (0, 0)),
            pl.BlockSpec((1, H), lambda i: (0, 0)),
            pl.BlockSpec((H, C_OUT), lambda i: (0, 0)),
            pl.BlockSpec((1, C_OUT), lambda i: (0, 0)),
        ],
        out_specs=pl.BlockSpec((BLK, C_OUT), lambda i: (i, 0)),
        out_shape=jax.ShapeDtypeStruct((N, C_OUT), jnp.float32),
    )(hsum2, ln0_g, ln0_b, w1, b1, ln1_g, ln1_b, w2, b2)


def kernel(features_0, edge_index, e_feat_org, fc_w, fc_b, ln0_g, ln0_b,
           w1, b1, ln1_g, ln1_b, w2, b2):
    src = edge_index[0]
    dst = edge_index[1]
    pad = E_PAD - E
    fill = (jnp.arange(pad, dtype=jnp.int32) % (NP16 - N)) + N
    srcv = jnp.concatenate([src, fill]).reshape(NS * CPT, ECH)
    dstv = jnp.concatenate([dst, fill]).reshape(NS * CPT, ECH)

    feat_pad = jnp.concatenate(
        [features_0, jnp.zeros((NP16 - N, D_IN), jnp.float32)])
    h0p = _tc_fc(feat_pad, fc_w, fc_b.reshape(1, H))
    hsum2, _ = _sc_mega(h0p, srcv, dstv)
    return _tc_head(hsum2, ln0_g.reshape(1, H), ln0_b.reshape(1, H),
                    w1, b1.reshape(1, H), ln1_g.reshape(1, H),
                    ln1_b.reshape(1, H), w2, b2.reshape(1, C_OUT))


# speedup vs baseline: 5.1967x; 1.1826x over previous
import functools

import jax
import jax.numpy as jnp
from jax import lax
from jax.experimental import pallas as pl
from jax.experimental.pallas import tpu as pltpu
from jax.experimental.pallas import tpu_sc as plsc

N = 10000
E = 320000
D_IN = 128
H = 128
C_OUT = 64
L = 4

NC = 2
NS = 16
NW = NC * NS

COLS = H // NC
ECH = 64
CPT = 320
QCH = 64
E_PAD = ECH * CPT * NS
NP16 = 10112
TROWS = NP16 // NS

BLK = 2000
BLKF = TROWS
GRID = N // BLK

_mesh = plsc.VectorSubcoreMesh(core_axis_name="c", subcore_axis_name="s")


def _sc_mega_body(h0p, srcv, dstv, hsum_out, t_flat,
                  idx_s, idx_d, rows0, rows1, rows2, no_v, ni_v, cc_v,
                  acc_sh, gsem0, gsem1, gsem2):
    c = lax.axis_index("c")
    s = lax.axis_index("s")
    base = s * TROWS
    NVH = H // 16

    zro16 = jnp.zeros((16,), jnp.float32)
    one16 = jnp.ones((16,), jnp.float32)
    mask0 = lax.broadcasted_iota(jnp.int32, (16,), 0) == 0

    def fill(buf, val16):
        @pl.loop(0, 64)
        def _f(i):
            for k in range(NVH):
                buf[i, pl.ds(k * 16, 16)] = val16

    def zero_acc_slice():
        @pl.loop(0, 9)
        def _z(k):
            pltpu.sync_copy(rows0.at[pl.ds(0, 64)],
                            acc_sh.at[pl.ds(base + k * 64, 64)])
        pltpu.sync_copy(rows0.at[pl.ds(0, 56)],
                        acc_sh.at[pl.ds(base + 576, 56)])

    def norm_block(lo, nr, p):
        pltpu.sync_copy(acc_sh.at[pl.ds(base + lo, nr)],
                        rows0.at[pl.ds(0, nr)])

        @pl.loop(0, nr)
        def _n(r):
            x = jnp.maximum(rows0[r, pl.ds(0, 16)], 1.0)
            iv = plsc.bitcast(x, jnp.int32)
            y = plsc.bitcast(jnp.int32(0x5F3759DF) - (iv >> 1), jnp.float32)
            for _ in range(3):
                y = y * (1.5 - 0.5 * x * y * y)
            ridx = jnp.full((16,), lo + r, jnp.int32)
            if p == 0:
                plsc.store_scatter(no_v, [ridx], y, mask=mask0)
            else:
                plsc.store_scatter(ni_v, [ridx], y, mask=mask0)
                nov = plsc.load_gather(no_v, [ridx])
                plsc.store_scatter(cc_v, [ridx], y * nov, mask=mask0)

    def stage_block(lo, nr):
        pltpu.sync_copy(h0p.at[pl.ds(base + lo, nr)], rows0.at[pl.ds(0, nr)])
        pltpu.sync_copy(rows0.at[pl.ds(0, nr)],
                        hsum_out.at[c, pl.ds(base + lo, nr)])

        @pl.loop(0, nr)
        def _t0(r):
            nov = plsc.load_gather(no_v, [jnp.full((16,), lo + r, jnp.int32)])
            for kk in range(NVH):
                sl = pl.ds(kk * 16, 16)
                rows0[r, sl] = rows0[r, sl] * nov

        pltpu.sync_copy(rows0.at[pl.ds(0, nr)],
                        t_flat.at[pl.ds(c * NP16 + base + lo, nr)])

    def edge_pass():
        coff = c * NP16

        @pl.loop(0, CPT // QCH)
        def _q(q):
            hb = s * CPT + q * QCH
            pltpu.sync_copy(srcv.at[pl.ds(hb, QCH)], idx_s)
            pltpu.sync_copy(dstv.at[pl.ds(hb, QCH)], idx_d)

            @pl.loop(0, QCH)
            def _off(r):
                for kk in range(ECH // 16):
                    sl = pl.ds(kk * 16, 16)
                    idx_s[r, sl] = idx_s[r, sl] + coff

            pltpu.make_async_copy(t_flat.at[idx_s.at[0]], rows0, gsem0).start()
            pltpu.make_async_copy(t_flat.at[idx_s.at[1]], rows1, gsem1).start()
            pltpu.make_async_copy(t_flat.at[idx_s.at[2]], rows2, gsem2).start()

            lanes = ((rows0, gsem0), (rows1, gsem1), (rows2, gsem2))

            @pl.loop(0, QCH - 1, step=3)
            def _edges(j0):
                for b, (rows, gsem) in enumerate(lanes):
                    j = j0 + b
                    pltpu.make_async_copy(
                        t_flat.at[idx_s.at[j]], rows, gsem).wait()
                    pltpu.sync_copy(rows, acc_sh.at[idx_d.at[j]], add=True)

                    @pl.when(j + 3 < QCH)
                    def _():
                        pltpu.make_async_copy(
                            t_flat.at[idx_s.at[j + 3]], rows, gsem).start()

            pltpu.make_async_copy(
                t_flat.at[idx_s.at[QCH - 1]], rows0, gsem0).wait()
            pltpu.sync_copy(rows0, acc_sh.at[idx_d.at[QCH - 1]], add=True)

    def scale_block(lo, nr, last):
        pltpu.sync_copy(acc_sh.at[pl.ds(base + lo, nr)],
                        rows0.at[pl.ds(0, nr)])
        pltpu.sync_copy(hsum_out.at[c, pl.ds(base + lo, nr)],
                        rows1.at[pl.ds(0, nr)])

        if last:
            @pl.loop(0, nr)
            def _upd_last(r):
                ridx = jnp.full((16,), lo + r, jnp.int32)
                niv = plsc.load_gather(ni_v, [ridx])
                for kk in range(NVH):
                    sl = pl.ds(kk * 16, 16)
                    rows1[r, sl] = rows1[r, sl] + niv * rows0[r, sl]
        else:
            @pl.loop(0, nr)
            def _upd(r):
                ridx = jnp.full((16,), lo + r, jnp.int32)
                niv = plsc.load_gather(ni_v, [ridx])
                ccv = plsc.load_gather(cc_v, [ridx])
                for kk in range(NVH):
                    sl = pl.ds(kk * 16, 16)
                    a = rows0[r, sl]
                    rows1[r, sl] = rows1[r, sl] + niv * a
                    rows0[r, sl] = ccv * a

        pltpu.sync_copy(rows1.at[pl.ds(0, nr)],
                        hsum_out.at[c, pl.ds(base + lo, nr)])
        if not last:
            pltpu.sync_copy(rows0.at[pl.ds(0, nr)],
                            t_flat.at[pl.ds(c * NP16 + base + lo, nr)])

    def scale_phase(last):
        @pl.loop(0, 9)
        def _sc(k):
            scale_block(k * 64, 64, last)

        scale_block(576, 56, last)
        if not last:
            fill(rows0, zro16)
            zero_acc_slice()

    fill(rows0, zro16)
    zero_acc_slice()
    plsc.subcore_barrier()

    for p in range(2):
        idxv = srcv if p == 0 else dstv
        fill(rows1, one16)

        @pl.loop(0, CPT // QCH)
        def _dq(q):
            pltpu.sync_copy(idxv.at[pl.ds(s * CPT + q * QCH, QCH)], idx_d)

            @pl.loop(0, QCH)
            def _deg_scatter(j):
                pltpu.sync_copy(rows1.at[pl.ds(0, ECH)],
                                acc_sh.at[idx_d.at[j]], add=True)

        plsc.subcore_barrier()

        @pl.loop(0, 9)
        def _nb(k):
            norm_block(k * 64, 64, p)

        norm_block(576, 56, p)

        fill(rows0, zro16)
        zero_acc_slice()
        plsc.subcore_barrier()

    @pl.loop(0, 9)
    def _stg(k):
        stage_block(k * 64, 64)

    stage_block(576, 56)
    plsc.subcore_barrier()

    @pl.loop(0, L - 1)
    def _layer(_):
        edge_pass()
        plsc.subcore_barrier()
        scale_phase(False)
        plsc.subcore_barrier()

    edge_pass()
    plsc.subcore_barrier()
    scale_phase(True)


_sc_mega = functools.partial(
    pl.kernel,
    out_type=[jax.ShapeDtypeStruct((NC, NP16, H), jnp.float32),
              jax.ShapeDtypeStruct((NC * NP16, H), jnp.float32)],
    mesh=_mesh,
    scratch_types=[
        pltpu.VMEM((QCH, ECH), jnp.int32),
        pltpu.VMEM((QCH, ECH), jnp.int32),
        pltpu.VMEM((ECH, H), jnp.float32),
        pltpu.VMEM((ECH, H), jnp.float32),
        pltpu.VMEM((ECH, H), jnp.float32),
        pltpu.VMEM((TROWS,), jnp.float32),
        pltpu.VMEM((TROWS,), jnp.float32),
        pltpu.VMEM((TROWS,), jnp.float32),
        pltpu.VMEM_SHARED((NP16, H), jnp.float32),
        pltpu.SemaphoreType.DMA,
        pltpu.SemaphoreType.DMA,
        pltpu.SemaphoreType.DMA,
    ],
    compiler_params=pltpu.CompilerParams(needs_layout_passes=False),
)(_sc_mega_body)


def _tc_fc_body(feat_ref, fcw_ref, fcb_ref, h0_ref):
    h0_ref[...] = jnp.dot(feat_ref[...], fcw_ref[...],
                          preferred_element_type=jnp.float32) + fcb_ref[...]


def _tc_fc(feat_pad, fc_w, fc_b):
    return pl.pallas_call(
        _tc_fc_body,
        grid=(NP16 // BLKF,),
        in_specs=[
            pl.BlockSpec((BLKF, D_IN), lambda i: (i, 0)),
            pl.BlockSpec((D_IN, H), lambda i: (0, 0)),
            pl.BlockSpec((1, H), lambda i: (0, 0)),
        ],
        out_specs=pl.BlockSpec((BLKF, H), lambda i: (i, 0)),
        out_shape=jax.ShapeDtypeStruct((NP16, H), jnp.float32),
    )(feat_pad, fc_w, fc_b)


def _tc_head_body(hs_ref, g0_ref, b0_ref, w1_ref, b1_ref, g1_ref, b1n_ref,
                  w2_ref, b2_ref, out_ref):
    hs = jnp.concatenate([hs_ref[0][:, :COLS], hs_ref[1][:, COLS:]], axis=1)
    x = hs * (1.0 / (L + 1.0))
    mu = jnp.mean(x, axis=1, keepdims=True)
    var = jnp.mean((x - mu) * (x - mu), axis=1, keepdims=True)
    x = (x - mu) * lax.rsqrt(var + 1e-5) * g0_ref[...] + b0_ref[...]
    y = jnp.dot(x, w1_ref[...], preferred_element_type=jnp.float32) + b1_ref[...]
    y = jnp.where(y > 0, y, jnp.exp(jnp.minimum(y, 0.0)) - 1.0)
    mu = jnp.mean(y, axis=1, keepdims=True)
    var = jnp.mean((y - mu) * (y - mu), axis=1, keepdims=True)
    y = (y - mu) * lax.rsqrt(var + 1e-5) * g1_ref[...] + b1n_ref[...]
    z = jnp.dot(y, w2_ref[...], preferred_element_type=jnp.float32) + b2_ref[...]
    nrm = jnp.sqrt(jnp.sum(z * z, axis=1, keepdims=True))
    out_ref[...] = z / jnp.maximum(nrm, 1e-12)


def _tc_head(hsum2, ln0_g, ln0_b, w1, b1, ln1_g, ln1_b, w2, b2):
    return pl.pallas_call(
        _tc_head_body,
        grid=(GRID,),
        in_specs=[
            pl.BlockSpec((NC, BLK, H), lambda i: (0, i, 0)),
            pl.BlockSpec((1, H), lambda i: (0, 0)),
            pl.BlockSpec((1, H), lambda i: (0, 0)),
            pl.BlockSpec((H, H), lambda i: (0, 0)),
            pl.BlockSpec((1, H), lambda i: (0, 0)),
            pl.BlockSpec((1, H), lambda i: (0, 0)),
            pl.BlockSpec((1, H), lambda i: (0, 0)),
            pl.BlockSpec((H, C_OUT), lambda i: (0, 0)),
            pl.BlockSpec((1, C_OUT), lambda i: (0, 0)),
        ],
        out_specs=pl.BlockSpec((BLK, C_OUT), lambda i: (i, 0)),
        out_shape=jax.ShapeDtypeStruct((N, C_OUT), jnp.float32),
    )(hsum2, ln0_g, ln0_b, w1, b1, ln1_g, ln1_b, w2, b2)


def kernel(features_0, edge_index, e_feat_org, fc_w, fc_b, ln0_g, ln0_b,
           w1, b1, ln1_g, ln1_b, w2, b2):
    src = edge_index[0]
    dst = edge_index[1]
    pad = E_PAD - E
    fill = (jnp.arange(pad, dtype=jnp.int32) % (NP16 - N)) + N
    srcv = jnp.concatenate([src, fill]).reshape(NS * CPT, ECH)
    dstv = jnp.concatenate([dst, fill]).reshape(NS * CPT, ECH)

    feat_pad = jnp.concatenate(
        [features_0, jnp.zeros((NP16 - N, D_IN), jnp.float32)])
    h0p = _tc_fc(feat_pad, fc_w, fc_b.reshape(1, H))
    hsum2, _ = _sc_mega(h0p, srcv, dstv)
    return _tc_head(hsum2, ln0_g.reshape(1, H), ln0_b.reshape(1, H),
                    w1, b1.reshape(1, H), ln1_g.reshape(1, H),
                    ln1_b.reshape(1, H), w2, b2.reshape(1, C_OUT))


# speedup vs baseline: 5.2487x; 1.0100x over previous
import functools

import jax
import jax.numpy as jnp
from jax import lax
from jax.experimental import pallas as pl
from jax.experimental.pallas import tpu as pltpu
from jax.experimental.pallas import tpu_sc as plsc

N = 10000
E = 320000
D_IN = 128
H = 128
C_OUT = 64
L = 4

NC = 2
NS = 16
NW = NC * NS

COLS = H // NC
ECH = 64
CPT = 320
QCH = 64
E_PAD = ECH * CPT * NS
NP16 = 10112
TROWS = NP16 // NS

BLK = 2000
BLKF = TROWS
GRID = N // BLK

_mesh = plsc.VectorSubcoreMesh(core_axis_name="c", subcore_axis_name="s")


def _sc_mega_body(h0p, srcv, dstv, hsum_out, t_flat,
                  idx_s, idx_d, rows0, rows1, rows2, no_v, ni_v, cc_v,
                  acc_sh, gsem0, gsem1, gsem2):
    c = lax.axis_index("c")
    s = lax.axis_index("s")
    base = s * TROWS
    NVH = H // 16

    zro16 = jnp.zeros((16,), jnp.float32)
    one16 = jnp.ones((16,), jnp.float32)
    mask0 = lax.broadcasted_iota(jnp.int32, (16,), 0) == 0

    def fill(buf, val16):
        @pl.loop(0, 64)
        def _f(i):
            for k in range(NVH):
                buf[i, pl.ds(k * 16, 16)] = val16

    def zero_acc_slice():
        @pl.loop(0, 9)
        def _z(k):
            pltpu.sync_copy(rows0.at[pl.ds(0, 64)],
                            acc_sh.at[pl.ds(base + k * 64, 64)])
        pltpu.sync_copy(rows0.at[pl.ds(0, 56)],
                        acc_sh.at[pl.ds(base + 576, 56)])

    def norm_block(lo, nr, p):
        pltpu.sync_copy(acc_sh.at[pl.ds(base + lo, nr)],
                        rows0.at[pl.ds(0, nr)])

        @pl.loop(0, nr)
        def _n(r):
            x = jnp.maximum(rows0[r, pl.ds(0, 16)], 1.0)
            iv = plsc.bitcast(x, jnp.int32)
            y = plsc.bitcast(jnp.int32(0x5F3759DF) - (iv >> 1), jnp.float32)
            for _ in range(3):
                y = y * (1.5 - 0.5 * x * y * y)
            ridx = jnp.full((16,), lo + r, jnp.int32)
            if p == 0:
                plsc.store_scatter(no_v, [ridx], y, mask=mask0)
            else:
                plsc.store_scatter(ni_v, [ridx], y, mask=mask0)
                nov = plsc.load_gather(no_v, [ridx])
                plsc.store_scatter(cc_v, [ridx], y * nov, mask=mask0)

    def stage_block(lo, nr):
        pltpu.sync_copy(h0p.at[pl.ds(base + lo, nr)], rows0.at[pl.ds(0, nr)])
        pltpu.sync_copy(rows0.at[pl.ds(0, nr)],
                        hsum_out.at[c, pl.ds(base + lo, nr)])

        @pl.loop(0, nr)
        def _t0(r):
            nov = plsc.load_gather(no_v, [jnp.full((16,), lo + r, jnp.int32)])
            for kk in range(NVH):
                sl = pl.ds(kk * 16, 16)
                rows0[r, sl] = rows0[r, sl] * nov

        pltpu.sync_copy(rows0.at[pl.ds(0, nr)],
                        t_flat.at[pl.ds(c * NP16 + base + lo, nr)])

    def edge_pass():
        coff = c * NP16

        @pl.loop(0, CPT // QCH)
        def _q(q):
            hb = s * CPT + q * QCH
            pltpu.sync_copy(srcv.at[pl.ds(hb, QCH)], idx_s)
            pltpu.sync_copy(dstv.at[pl.ds(hb, QCH)], idx_d)

            @pl.loop(0, QCH)
            def _off(r):
                for kk in range(ECH // 16):
                    sl = pl.ds(kk * 16, 16)
                    idx_s[r, sl] = idx_s[r, sl] + coff

            pltpu.make_async_copy(t_flat.at[idx_s.at[0]], rows0, gsem0).start()
            pltpu.make_async_copy(t_flat.at[idx_s.at[1]], rows1, gsem1).start()
            pltpu.make_async_copy(t_flat.at[idx_s.at[2]], rows2, gsem2).start()

            lanes = ((rows0, gsem0), (rows1, gsem1), (rows2, gsem2))

            @pl.loop(0, QCH - 1, step=3)
            def _edges(j0):
                for b, (rows, gsem) in enumerate(lanes):
                    j = j0 + b
                    pltpu.make_async_copy(
                        t_flat.at[idx_s.at[j]], rows, gsem).wait()
                    pltpu.sync_copy(rows, acc_sh.at[idx_d.at[j]], add=True)

                    @pl.when(j + 3 < QCH)
                    def _():
                        pltpu.make_async_copy(
                            t_flat.at[idx_s.at[j + 3]], rows, gsem).start()

            pltpu.make_async_copy(
                t_flat.at[idx_s.at[QCH - 1]], rows0, gsem0).wait()
            pltpu.sync_copy(rows0, acc_sh.at[idx_d.at[QCH - 1]], add=True)

    def scale_block(lo, nr, last):
        pltpu.sync_copy(acc_sh.at[pl.ds(base + lo, nr)],
                        rows0.at[pl.ds(0, nr)])
        pltpu.sync_copy(hsum_out.at[c, pl.ds(base + lo, nr)],
                        rows1.at[pl.ds(0, nr)])

        if last:
            @pl.loop(0, nr)
            def _upd_last(r):
                ridx = jnp.full((16,), lo + r, jnp.int32)
                niv = plsc.load_gather(ni_v, [ridx])
                for kk in range(NVH):
                    sl = pl.ds(kk * 16, 16)
                    rows1[r, sl] = rows1[r, sl] + niv * rows0[r, sl]
        else:
            @pl.loop(0, nr)
            def _upd(r):
                ridx = jnp.full((16,), lo + r, jnp.int32)
                niv = plsc.load_gather(ni_v, [ridx])
                ccv = plsc.load_gather(cc_v, [ridx])
                for kk in range(NVH):
                    sl = pl.ds(kk * 16, 16)
                    a = rows0[r, sl]
                    rows1[r, sl] = rows1[r, sl] + niv * a
                    rows0[r, sl] = ccv * a

        pltpu.sync_copy(rows1.at[pl.ds(0, nr)],
                        hsum_out.at[c, pl.ds(base + lo, nr)])
        if not last:
            pltpu.sync_copy(rows0.at[pl.ds(0, nr)],
                            t_flat.at[pl.ds(c * NP16 + base + lo, nr)])

    def scale_phase(last):
        @pl.loop(0, 9)
        def _sc(k):
            scale_block(k * 64, 64, last)

        scale_block(576, 56, last)
        if not last:
            fill(rows0, zro16)
            zero_acc_slice()

    fill(rows0, zro16)
    zero_acc_slice()
    plsc.subcore_barrier()

    for p in range(2):
        idxv = srcv if p == 0 else dstv
        fill(rows1, one16)

        @pl.loop(0, CPT // QCH)
        def _dq(q):
            pltpu.sync_copy(idxv.at[pl.ds(s * CPT + q * QCH, QCH)], idx_d)

            @pl.loop(0, QCH)
            def _deg_scatter(j):
                @pl.when(j >= 2)
                def _():
                    pltpu.make_async_copy(
                        rows1.at[pl.ds(0, ECH)],
                        acc_sh.at[idx_d.at[j - 2]], gsem0).wait()
                pltpu.async_copy(
                    rows1.at[pl.ds(0, ECH)],
                    acc_sh.at[idx_d.at[j]], gsem0, add=True)

            pltpu.make_async_copy(
                rows1.at[pl.ds(0, ECH)],
                acc_sh.at[idx_d.at[QCH - 2]], gsem0).wait()
            pltpu.make_async_copy(
                rows1.at[pl.ds(0, ECH)],
                acc_sh.at[idx_d.at[QCH - 1]], gsem0).wait()

        plsc.subcore_barrier()

        @pl.loop(0, 9)
        def _nb(k):
            norm_block(k * 64, 64, p)

        norm_block(576, 56, p)

        fill(rows0, zro16)
        zero_acc_slice()
        plsc.subcore_barrier()

    @pl.loop(0, 9)
    def _stg(k):
        stage_block(k * 64, 64)

    stage_block(576, 56)
    plsc.subcore_barrier()

    @pl.loop(0, L - 1)
    def _layer(_):
        edge_pass()
        plsc.subcore_barrier()
        scale_phase(False)
        plsc.subcore_barrier()

    edge_pass()
    plsc.subcore_barrier()
    scale_phase(True)


_sc_mega = functools.partial(
    pl.kernel,
    out_type=[jax.ShapeDtypeStruct((NC, NP16, H), jnp.float32),
              jax.ShapeDtypeStruct((NC * NP16, H), jnp.float32)],
    mesh=_mesh,
    scratch_types=[
        pltpu.VMEM((QCH, ECH), jnp.int32),
        pltpu.VMEM((QCH, ECH), jnp.int32),
        pltpu.VMEM((ECH, H), jnp.float32),
        pltpu.VMEM((ECH, H), jnp.float32),
        pltpu.VMEM((ECH, H), jnp.float32),
        pltpu.VMEM((TROWS,), jnp.float32),
        pltpu.VMEM((TROWS,), jnp.float32),
        pltpu.VMEM((TROWS,), jnp.float32),
        pltpu.VMEM_SHARED((NP16, H), jnp.float32),
        pltpu.SemaphoreType.DMA,
        pltpu.SemaphoreType.DMA,
        pltpu.SemaphoreType.DMA,
    ],
    compiler_params=pltpu.CompilerParams(needs_layout_passes=False),
)(_sc_mega_body)


def _tc_fc_body(feat_ref, fcw_ref, fcb_ref, h0_ref):
    h0_ref[...] = jnp.dot(feat_ref[...], fcw_ref[...],
                          preferred_element_type=jnp.float32) + fcb_ref[...]


def _tc_fc(feat_pad, fc_w, fc_b):
    return pl.pallas_call(
        _tc_fc_body,
        grid=(NP16 // BLKF,),
        in_specs=[
            pl.BlockSpec((BLKF, D_IN), lambda i: (i, 0)),
            pl.BlockSpec((D_IN, H), lambda i: (0, 0)),
            pl.BlockSpec((1, H), lambda i: (0, 0)),
        ],
        out_specs=pl.BlockSpec((BLKF, H), lambda i: (i, 0)),
        out_shape=jax.ShapeDtypeStruct((NP16, H), jnp.float32),
    )(feat_pad, fc_w, fc_b)


def _tc_head_body(hs_ref, g0_ref, b0_ref, w1_ref, b1_ref, g1_ref, b1n_ref,
                  w2_ref, b2_ref, out_ref):
    hs = jnp.concatenate([hs_ref[0][:, :COLS], hs_ref[1][:, COLS:]], axis=1)
    x = hs * (1.0 / (L + 1.0))
    mu = jnp.mean(x, axis=1, keepdims=True)
    var = jnp.mean((x - mu) * (x - mu), axis=1, keepdims=True)
    x = (x - mu) * lax.rsqrt(var + 1e-5) * g0_ref[...] + b0_ref[...]
    y = jnp.dot(x, w1_ref[...], preferred_element_type=jnp.float32) + b1_ref[...]
    y = jnp.where(y > 0, y, jnp.exp(jnp.minimum(y, 0.0)) - 1.0)
    mu = jnp.mean(y, axis=1, keepdims=True)
    var = jnp.mean((y - mu) * (y - mu), axis=1, keepdims=True)
    y = (y - mu) * lax.rsqrt(var + 1e-5) * g1_ref[...] + b1n_ref[...]
    z = jnp.dot(y, w2_ref[...], preferred_element_type=jnp.float32) + b2_ref[...]
    nrm = jnp.sqrt(jnp.sum(z * z, axis=1, keepdims=True))
    out_ref[...] = z / jnp.maximum(nrm, 1e-12)


def _tc_head(hsum2, ln0_g, ln0_b, w1, b1, ln1_g, ln1_b, w2, b2):
    return pl.pallas_call(
        _tc_head_body,
        grid=(GRID,),
        in_specs=[
            pl.BlockSpec((NC, BLK, H), lambda i: (0, i, 0)),
            pl.BlockSpec((1, H), lambda i: (0, 0)),
            pl.BlockSpec((1, H), lambda i: (0, 0)),
            pl.BlockSpec((H, H), lambda i: (0, 0)),
            pl.BlockSpec((1, H), lambda i: (0, 0)),
            pl.BlockSpec((1, H), lambda i: (0, 0)),
            pl.BlockSpec((1, H), lambda i: (0, 0)),
            pl.BlockSpec((H, C_OUT), lambda i: (0, 0)),
            pl.BlockSpec((1, C_OUT), lambda i: (0, 0)),
        ],
        out_specs=pl.BlockSpec((BLK, C_OUT), lambda i: (i, 0)),
        out_shape=jax.ShapeDtypeStruct((N, C_OUT), jnp.float32),
    )(hsum2, ln0_g, ln0_b, w1, b1, ln1_g, ln1_b, w2, b2)


def kernel(features_0, edge_index, e_feat_org, fc_w, fc_b, ln0_g, ln0_b,
           w1, b1, ln1_g, ln1_b, w2, b2):
    src = edge_index[0]
    dst = edge_index[1]
    pad = E_PAD - E
    fill = (jnp.arange(pad, dtype=jnp.int32) % (NP16 - N)) + N
    srcv = jnp.concatenate([src, fill]).reshape(NS * CPT, ECH)
    dstv = jnp.concatenate([dst, fill]).reshape(NS * CPT, ECH)

    feat_pad = jnp.concatenate(
        [features_0, jnp.zeros((NP16 - N, D_IN), jnp.float32)])
    h0p = _tc_fc(feat_pad, fc_w, fc_b.reshape(1, H))
    hsum2, _ = _sc_mega(h0p, srcv, dstv)
    return _tc_head(hsum2, ln0_g.reshape(1, H), ln0_b.reshape(1, H),
                    w1, b1.reshape(1, H), ln1_g.reshape(1, H),
                    ln1_b.reshape(1, H), w2, b2.reshape(1, C_OUT))


# speedup vs baseline: 5.2547x; 1.0011x over previous
import functools

import jax
import jax.numpy as jnp
from jax import lax
from jax.experimental import pallas as pl
from jax.experimental.pallas import tpu as pltpu
from jax.experimental.pallas import tpu_sc as plsc

N = 10000
E = 320000
D_IN = 128
H = 128
C_OUT = 64
L = 4

NC = 2
NS = 16
NW = NC * NS

COLS = H // NC
ECH = 64
CPT = 320
QCH = 64
E_PAD = ECH * CPT * NS
NP16 = 10112
TROWS = NP16 // NS

BLK = 2000
BLKF = TROWS
GRID = N // BLK

_mesh = plsc.VectorSubcoreMesh(core_axis_name="c", subcore_axis_name="s")


def _sc_mega_body(h0p, srcv, dstv, hsum_out, t_flat,
                  idx_s, idx_d, rows0, rows1, rows2, no_v, ni_v, cc_v,
                  acc_sh, gsem0, gsem1, gsem2):
    c = lax.axis_index("c")
    s = lax.axis_index("s")
    base = s * TROWS
    NVH = H // 16

    zro16 = jnp.zeros((16,), jnp.float32)
    one16 = jnp.ones((16,), jnp.float32)
    mask0 = lax.broadcasted_iota(jnp.int32, (16,), 0) == 0

    def fill(buf, val16):
        @pl.loop(0, 64)
        def _f(i):
            for k in range(NVH):
                buf[i, pl.ds(k * 16, 16)] = val16

    def zero_acc_slice():
        @pl.loop(0, 9)
        def _z(k):
            pltpu.sync_copy(rows0.at[pl.ds(0, 64)],
                            acc_sh.at[pl.ds(base + k * 64, 64)])
        pltpu.sync_copy(rows0.at[pl.ds(0, 56)],
                        acc_sh.at[pl.ds(base + 576, 56)])

    def norm_block(lo, nr, p):
        pltpu.sync_copy(acc_sh.at[pl.ds(base + lo, nr)],
                        rows0.at[pl.ds(0, nr)])

        @pl.loop(0, nr)
        def _n(r):
            x = jnp.maximum(rows0[r, pl.ds(0, 16)], 1.0)
            iv = plsc.bitcast(x, jnp.int32)
            y = plsc.bitcast(jnp.int32(0x5F3759DF) - (iv >> 1), jnp.float32)
            for _ in range(3):
                y = y * (1.5 - 0.5 * x * y * y)
            ridx = jnp.full((16,), lo + r, jnp.int32)
            if p == 0:
                plsc.store_scatter(no_v, [ridx], y, mask=mask0)
            else:
                plsc.store_scatter(ni_v, [ridx], y, mask=mask0)
                nov = plsc.load_gather(no_v, [ridx])
                plsc.store_scatter(cc_v, [ridx], y * nov, mask=mask0)

    def stage_block(lo, nr):
        pltpu.sync_copy(h0p.at[pl.ds(base + lo, nr)], rows0.at[pl.ds(0, nr)])
        pltpu.sync_copy(rows0.at[pl.ds(0, nr)],
                        hsum_out.at[c, pl.ds(base + lo, nr)])

        @pl.loop(0, nr)
        def _t0(r):
            nov = plsc.load_gather(no_v, [jnp.full((16,), lo + r, jnp.int32)])
            for kk in range(NVH):
                sl = pl.ds(kk * 16, 16)
                rows0[r, sl] = rows0[r, sl] * nov

        pltpu.sync_copy(rows0.at[pl.ds(0, nr)],
                        t_flat.at[pl.ds(c * NP16 + base + lo, nr)])

    def edge_pass():
        coff = c * NP16

        @pl.loop(0, CPT // QCH)
        def _q(q):
            hb = s * CPT + q * QCH
            pltpu.sync_copy(srcv.at[pl.ds(hb, QCH)], idx_s)
            pltpu.sync_copy(dstv.at[pl.ds(hb, QCH)], idx_d)

            @pl.loop(0, QCH)
            def _off(r):
                for kk in range(ECH // 16):
                    sl = pl.ds(kk * 16, 16)
                    idx_s[r, sl] = idx_s[r, sl] + coff

            pltpu.make_async_copy(t_flat.at[idx_s.at[0]], rows0, gsem0).start()
            pltpu.make_async_copy(t_flat.at[idx_s.at[1]], rows1, gsem1).start()
            pltpu.make_async_copy(t_flat.at[idx_s.at[2]], rows2, gsem2).start()

            lanes = ((rows0, gsem0), (rows1, gsem1), (rows2, gsem2))

            @pl.loop(0, QCH - 1, step=3)
            def _edges(j0):
                for b, (rows, gsem) in enumerate(lanes):
                    j = j0 + b
                    pltpu.make_async_copy(
                        t_flat.at[idx_s.at[j]], rows, gsem).wait()
                    pltpu.sync_copy(rows, acc_sh.at[idx_d.at[j]], add=True)

                    @pl.when(j + 3 < QCH)
                    def _():
                        pltpu.make_async_copy(
                            t_flat.at[idx_s.at[j + 3]], rows, gsem).start()

            pltpu.make_async_copy(
                t_flat.at[idx_s.at[QCH - 1]], rows0, gsem0).wait()
            pltpu.sync_copy(rows0, acc_sh.at[idx_d.at[QCH - 1]], add=True)

    def scale_block(lo, nr, last):
        pltpu.sync_copy(acc_sh.at[pl.ds(base + lo, nr)],
                        rows0.at[pl.ds(0, nr)])
        pltpu.sync_copy(hsum_out.at[c, pl.ds(base + lo, nr)],
                        rows1.at[pl.ds(0, nr)])

        if last:
            @pl.loop(0, nr)
            def _upd_last(r):
                ridx = jnp.full((16,), lo + r, jnp.int32)
                niv = plsc.load_gather(ni_v, [ridx])
                for kk in range(NVH):
                    sl = pl.ds(kk * 16, 16)
                    rows1[r, sl] = rows1[r, sl] + niv * rows0[r, sl]
        else:
            @pl.loop(0, nr)
            def _upd(r):
                ridx = jnp.full((16,), lo + r, jnp.int32)
                niv = plsc.load_gather(ni_v, [ridx])
                ccv = plsc.load_gather(cc_v, [ridx])
                for kk in range(NVH):
                    sl = pl.ds(kk * 16, 16)
                    a = rows0[r, sl]
                    rows1[r, sl] = rows1[r, sl] + niv * a
                    rows0[r, sl] = ccv * a

        pltpu.sync_copy(rows1.at[pl.ds(0, nr)],
                        hsum_out.at[c, pl.ds(base + lo, nr)])
        if not last:
            pltpu.sync_copy(rows0.at[pl.ds(0, nr)],
                            t_flat.at[pl.ds(c * NP16 + base + lo, nr)])

    def scale_phase(last):
        @pl.loop(0, 9)
        def _sc(k):
            scale_block(k * 64, 64, last)

        scale_block(576, 56, last)
        if not last:
            fill(rows0, zro16)
            zero_acc_slice()

    fill(rows0, zro16)
    zero_acc_slice()
    plsc.subcore_barrier()

    for p in range(2):
        idxv = srcv if p == 0 else dstv
        fill(rows1, one16)

        @pl.loop(0, CPT // QCH)
        def _dq(q):
            pltpu.sync_copy(idxv.at[pl.ds(s * CPT + q * QCH, QCH)], idx_d)

            @pl.loop(0, QCH)
            def _deg_scatter(j):
                @pl.when(j >= 6)
                def _():
                    pltpu.make_async_copy(
                        rows1.at[pl.ds(0, ECH)],
                        acc_sh.at[idx_d.at[j - 6]], gsem0).wait()
                pltpu.async_copy(
                    rows1.at[pl.ds(0, ECH)],
                    acc_sh.at[idx_d.at[j]], gsem0, add=True)

            @pl.loop(0, 6)
            def _deg_drain(j):
                pltpu.make_async_copy(
                    rows1.at[pl.ds(0, ECH)],
                    acc_sh.at[idx_d.at[QCH - 6 + j]], gsem0).wait()

        plsc.subcore_barrier()

        @pl.loop(0, 9)
        def _nb(k):
            norm_block(k * 64, 64, p)

        norm_block(576, 56, p)

        fill(rows0, zro16)
        zero_acc_slice()
        plsc.subcore_barrier()

    @pl.loop(0, 9)
    def _stg(k):
        stage_block(k * 64, 64)

    stage_block(576, 56)
    plsc.subcore_barrier()

    @pl.loop(0, L - 1)
    def _layer(_):
        edge_pass()
        plsc.subcore_barrier()
        scale_phase(False)
        plsc.subcore_barrier()

    edge_pass()
    plsc.subcore_barrier()
    scale_phase(True)


_sc_mega = functools.partial(
    pl.kernel,
    out_type=[jax.ShapeDtypeStruct((NC, NP16, H), jnp.float32),
              jax.ShapeDtypeStruct((NC * NP16, H), jnp.float32)],
    mesh=_mesh,
    scratch_types=[
        pltpu.VMEM((QCH, ECH), jnp.int32),
        pltpu.VMEM((QCH, ECH), jnp.int32),
        pltpu.VMEM((ECH, H), jnp.float32),
        pltpu.VMEM((ECH, H), jnp.float32),
        pltpu.VMEM((ECH, H), jnp.float32),
        pltpu.VMEM((TROWS,), jnp.float32),
        pltpu.VMEM((TROWS,), jnp.float32),
        pltpu.VMEM((TROWS,), jnp.float32),
        pltpu.VMEM_SHARED((NP16, H), jnp.float32),
        pltpu.SemaphoreType.DMA,
        pltpu.SemaphoreType.DMA,
        pltpu.SemaphoreType.DMA,
    ],
    compiler_params=pltpu.CompilerParams(needs_layout_passes=False),
)(_sc_mega_body)


def _tc_fc_body(feat_ref, fcw_ref, fcb_ref, h0_ref):
    h0_ref[...] = jnp.dot(feat_ref[...], fcw_ref[...],
                          preferred_element_type=jnp.float32) + fcb_ref[...]


def _tc_fc(feat_pad, fc_w, fc_b):
    return pl.pallas_call(
        _tc_fc_body,
        grid=(NP16 // BLKF,),
        in_specs=[
            pl.BlockSpec((BLKF, D_IN), lambda i: (i, 0)),
            pl.BlockSpec((D_IN, H), lambda i: (0, 0)),
            pl.BlockSpec((1, H), lambda i: (0, 0)),
        ],
        out_specs=pl.BlockSpec((BLKF, H), lambda i: (i, 0)),
        out_shape=jax.ShapeDtypeStruct((NP16, H), jnp.float32),
    )(feat_pad, fc_w, fc_b)


def _tc_head_body(hs_ref, g0_ref, b0_ref, w1_ref, b1_ref, g1_ref, b1n_ref,
                  w2_ref, b2_ref, out_ref):
    hs = jnp.concatenate([hs_ref[0][:, :COLS], hs_ref[1][:, COLS:]], axis=1)
    x = hs * (1.0 / (L + 1.0))
    mu = jnp.mean(x, axis=1, keepdims=True)
    var = jnp.mean((x - mu) * (x - mu), axis=1, keepdims=True)
    x = (x - mu) * lax.rsqrt(var + 1e-5) * g0_ref[...] + b0_ref[...]
    y = jnp.dot(x, w1_ref[...], preferred_element_type=jnp.float32) + b1_ref[...]
    y = jnp.where(y > 0, y, jnp.exp(jnp.minimum(y, 0.0)) - 1.0)
    mu = jnp.mean(y, axis=1, keepdims=True)
    var = jnp.mean((y - mu) * (y - mu), axis=1, keepdims=True)
    y = (y - mu) * lax.rsqrt(var + 1e-5) * g1_ref[...] + b1n_ref[...]
    z = jnp.dot(y, w2_ref[...], preferred_element_type=jnp.float32) + b2_ref[...]
    nrm = jnp.sqrt(jnp.sum(z * z, axis=1, keepdims=True))
    out_ref[...] = z / jnp.maximum(nrm, 1e-12)


def _tc_head(hsum2, ln0_g, ln0_b, w1, b1, ln1_g, ln1_b, w2, b2):
    return pl.pallas_call(
        _tc_head_body,
        grid=(GRID,),
        in_specs=[
            pl.BlockSpec((NC, BLK, H), lambda i: (0, i, 0)),
            pl.BlockSpec((1, H), lambda i: (0, 0)),
            pl.BlockSpec((1, H), lambda i: (0, 0)),
            pl.BlockSpec((H, H), lambda i: (0, 0)),
            pl.BlockSpec((1, H), lambda i: (0, 0)),
            pl.BlockSpec((1, H), lambda i: (0, 0)),
            pl.BlockSpec((1, H), lambda i: (0, 0)),
            pl.BlockSpec((H, C_OUT), lambda i: (0, 0)),
            pl.BlockSpec((1, C_OUT), lambda i: (0, 0)),
        ],
        out_specs=pl.BlockSpec((BLK, C_OUT), lambda i: (i, 0)),
        out_shape=jax.ShapeDtypeStruct((N, C_OUT), jnp.float32),
    )(hsum2, ln0_g, ln0_b, w1, b1, ln1_g, ln1_b, w2, b2)


def kernel(features_0, edge_index, e_feat_org, fc_w, fc_b, ln0_g, ln0_b,
           w1, b1, ln1_g, ln1_b, w2, b2):
    src = edge_index[0]
    dst = edge_index[1]
    pad = E_PAD - E
    fill = (jnp.arange(pad, dtype=jnp.int32) % (NP16 - N)) + N
    srcv = jnp.concatenate([src, fill]).reshape(NS * CPT, ECH)
    dstv = jnp.concatenate([dst, fill]).reshape(NS * CPT, ECH)

    feat_pad = jnp.concatenate(
        [features_0, jnp.zeros((NP16 - N, D_IN), jnp.float32)])
    h0p = _tc_fc(feat_pad, fc_w, fc_b.reshape(1, H))
    hsum2, _ = _sc_mega(h0p, srcv, dstv)
    return _tc_head(hsum2, ln0_g.reshape(1, H), ln0_b.reshape(1, H),
                    w1, b1.reshape(1, H), ln1_g.reshape(1, H),
                    ln1_b.reshape(1, H), w2, b2.reshape(1, C_OUT))
